# baseline (speedup 1.0000x reference)
import functools

import jax
import jax.numpy as jnp
from jax.experimental import pallas as pl
from jax.experimental.pallas import tpu as pltpu

H = 4096
T = 8192
EPS = 1e-06
BLK = 16
FP8_MAX = 448.0

BT = 256
BN = 1024
NB = H // BN


def _roll_l(x, s):
    return jnp.concatenate([x[:, s:], x[:, :s]], axis=1)


def _roll_r(x, s):
    return jnp.concatenate([x[:, -s:], x[:, :-s]], axis=1)


def _seg16_max(a, lane_mod16):
    m = a
    for s in (1, 2, 4, 8):
        r = _roll_l(m, s)
        m = jnp.where(lane_mod16 < 16 - s, jnp.maximum(m, r), m)
    for s in (1, 2, 4, 8):
        r = _roll_r(m, s)
        m = jnp.where(lane_mod16 >= s, jnp.maximum(m, r), m)
    return m


def _fp8_e4m3(x):
    return x.astype(jnp.float8_e4m3fn).astype(jnp.float32)


def _fp4_round_mag(a):
    lo = jnp.floor(a + a + 0.5) * 0.5
    mid = jnp.floor(a + 0.5)
    hi = jnp.where(a >= 5.0, 6.0, 4.0)
    return jnp.where(a < 2.0, lo, jnp.where(a < 4.0, mid, hi))


def _quant_dequant_bf16(x, gs, lane_mod16):
    amax = _seg16_max(jnp.abs(x), lane_mod16)
    sc = jnp.clip(amax * (gs * (1.0 / 6.0)), 0.0, FP8_MAX)
    sc8 = _fp8_e4m3(sc)
    ok = sc8 > 0.0
    t = jnp.where(ok, (x * gs) / jnp.where(ok, sc8, 1.0), 0.0)
    qm = _fp4_round_mag(jnp.clip(jnp.abs(t), 0.0, 6.0))
    q = jnp.where(t < 0.0, -qm, qm)
    return (q * sc8).astype(jnp.bfloat16)


PBN = 512


def _prep_body(wg_ref, w_ref, o_ref):
    g = pl.program_id(0)
    i = g // (H // PBN)
    x = w_ref[0]
    lane = jax.lax.broadcasted_iota(jnp.int32, x.shape, 1) % BLK
    o_ref[0] = _quant_dequant_bf16(x, wg_ref[i], lane)


def _prep_weights(w, wgscale):
    wdq = pl.pallas_call(
        _prep_body,
        grid=(3 * (H // PBN),),
        in_specs=[
            pl.BlockSpec(memory_space=pltpu.SMEM),
            pl.BlockSpec((1, PBN, H), lambda g: (g // (H // PBN), g % (H // PBN), 0)),
        ],
        out_specs=pl.BlockSpec((1, PBN, H), lambda g: (g // (H // PBN), g % (H // PBN), 0)),
        out_shape=jax.ShapeDtypeStruct((3, H, H), jnp.bfloat16),
        compiler_params=pltpu.CompilerParams(
            dimension_semantics=("parallel",),
            vmem_limit_bytes=100 * 1024 * 1024,
        ),
    )(wgscale, w)
    return jnp.swapaxes(wdq, 1, 2)


def _main_body(ag_ref, alpha_ref, hs_ref, wdq_ref, nw0_ref, nwc_ref, o_ref,
               resid_ref, a_ref):
    i = pl.program_id(1)
    n = pl.program_id(2)
    lane = jax.lax.broadcasted_iota(jnp.int32, (BT, H), 1) % BLK

    @pl.when(jnp.logical_and(i == 0, n == 0))
    def _():
        x = jnp.maximum(hs_ref[...], 0.0)
        resid_ref[...] = x
        ms = jnp.mean(x * x, axis=-1, keepdims=True)
        y = x * jax.lax.rsqrt(ms + EPS) * nw0_ref[0]
        a_ref[...] = _quant_dequant_bf16(y, ag_ref[0], lane)

    z = jax.lax.dot_general(
        a_ref[...], wdq_ref[0],
        dimension_numbers=(((1,), (0,)), ((), ())),
        preferred_element_type=jnp.float32,
    )
    off = pl.multiple_of(n * BN, BN)
    resid_ref[:, pl.ds(off, BN)] = resid_ref[:, pl.ds(off, BN)] + z * alpha_ref[i]

    @pl.when(n == NB - 1)
    def _():
        x = resid_ref[...]
        ms = jnp.mean(x * x, axis=-1, keepdims=True)
        y = x * jax.lax.rsqrt(ms + EPS) * nwc_ref[0]

        @pl.when(i < 2)
        def _():
            a_ref[...] = _quant_dequant_bf16(y, ag_ref[i + 1], lane)

        @pl.when(i == 2)
        def _():
            o_ref[...] = y


def kernel(hidden_states, norm_w, w, agscale, wgscale):
    wdq = _prep_weights(w, wgscale)
    alpha = 1.0 / (wgscale * agscale)
    norm_w3 = norm_w.reshape(4, 1, H)
    return pl.pallas_call(
        _main_body,
        grid=(T // BT, 3, NB),
        in_specs=[
            pl.BlockSpec(memory_space=pltpu.SMEM),
            pl.BlockSpec(memory_space=pltpu.SMEM),
            pl.BlockSpec((BT, H), lambda t, i, n: (t, 0)),
            pl.BlockSpec((1, H, BN), lambda t, i, n: (i, 0, n)),
            pl.BlockSpec((1, 1, H), lambda t, i, n: (0, 0, 0)),
            pl.BlockSpec((1, 1, H), lambda t, i, n: (i + 1, 0, 0)),
        ],
        out_specs=pl.BlockSpec((BT, H), lambda t, i, n: (t, 0)),
        out_shape=jax.ShapeDtypeStruct((T, H), jnp.float32),
        scratch_shapes=[
            pltpu.VMEM((BT, H), jnp.float32),
            pltpu.VMEM((BT, H), jnp.bfloat16),
        ],
        compiler_params=pltpu.CompilerParams(
            dimension_semantics=("parallel", "arbitrary", "arbitrary"),
            vmem_limit_bytes=100 * 1024 * 1024,
        ),
    )(agscale, alpha, hidden_states, wdq, norm_w3, norm_w3)


# speedup vs baseline: 1.3262x; 1.3262x over previous
import jax
import jax.numpy as jnp
from jax.experimental import pallas as pl
from jax.experimental.pallas import tpu as pltpu

H = 4096
T = 8192
EPS = 1e-06
BLK = 16
FP8_MAX = 448.0

BT = 256
BN = 1024
NB = H // BN


def _fp8_e4m3(x):
    return x.astype(jnp.float8_e4m3fn).astype(jnp.float32)


def _fp4_round_mag(a):
    lo = jnp.floor(a + a + 0.5) * 0.5
    mid = jnp.floor(a + 0.5)
    hi = jnp.where(a >= 5.0, 6.0, 4.0)
    return jnp.where(a < 2.0, lo, jnp.where(a < 4.0, mid, hi))


def _qd_core(x, gs):
    amax = jnp.max(jnp.abs(x), axis=-2, keepdims=True)
    sc = jnp.clip(amax * (gs * (1.0 / 6.0)), 0.0, FP8_MAX)
    sc8 = _fp8_e4m3(sc)
    ok = sc8 > 0.0
    t = jnp.where(ok, (x * gs) / jnp.where(ok, sc8, 1.0), 0.0)
    qm = _fp4_round_mag(jnp.clip(jnp.abs(t), 0.0, 6.0))
    q = jnp.where(t < 0.0, -qm, qm)
    return q * sc8


def _qd_cols_bf16(xt, gs):
    xr = xt.reshape(H // BLK, BLK, BT)
    return _qd_core(xr, gs).reshape(H, BT).astype(jnp.bfloat16)


PBN = 512


def _roll_l(x, s):
    return jnp.concatenate([x[:, s:], x[:, :s]], axis=1)


def _roll_r(x, s):
    return jnp.concatenate([x[:, -s:], x[:, :-s]], axis=1)


def _seg16_max_lanes(a, lane_mod16):
    m = a
    for s in (1, 2, 4, 8):
        r = _roll_l(m, s)
        m = jnp.where(lane_mod16 < 16 - s, jnp.maximum(m, r), m)
    for s in (1, 2, 4, 8):
        r = _roll_r(m, s)
        m = jnp.where(lane_mod16 >= s, jnp.maximum(m, r), m)
    return m


def _prep_body(wg_ref, w_ref, o_ref):
    g = pl.program_id(0)
    i = g // (H // PBN)
    gs = wg_ref[i]
    x = w_ref[0]
    lane = jax.lax.broadcasted_iota(jnp.int32, x.shape, 1) % BLK
    amax = _seg16_max_lanes(jnp.abs(x), lane)
    sc = jnp.clip(amax * (gs * (1.0 / 6.0)), 0.0, FP8_MAX)
    sc8 = _fp8_e4m3(sc)
    ok = sc8 > 0.0
    t = jnp.where(ok, (x * gs) / jnp.where(ok, sc8, 1.0), 0.0)
    qm = _fp4_round_mag(jnp.clip(jnp.abs(t), 0.0, 6.0))
    q = jnp.where(t < 0.0, -qm, qm)
    o_ref[0] = (q * sc8).astype(jnp.bfloat16)


def _prep_weights(w, wgscale):
    return pl.pallas_call(
        _prep_body,
        grid=(3 * (H // PBN),),
        in_specs=[
            pl.BlockSpec(memory_space=pltpu.SMEM),
            pl.BlockSpec((1, PBN, H), lambda g: (g // (H // PBN), g % (H // PBN), 0)),
        ],
        out_specs=pl.BlockSpec((1, PBN, H), lambda g: (g // (H // PBN), g % (H // PBN), 0)),
        out_shape=jax.ShapeDtypeStruct((3, H, H), jnp.bfloat16),
        compiler_params=pltpu.CompilerParams(
            dimension_semantics=("parallel",),
            vmem_limit_bytes=100 * 1024 * 1024,
        ),
    )(wgscale, w)


def _rms_scale(xt):
    ms = jnp.sum(xt * xt, axis=0, keepdims=True) * (1.0 / H)
    return jax.lax.rsqrt(ms + EPS)


def _main_body(ag_ref, alpha_ref, hst_ref, wdq_ref, nw0_ref, nwc_ref, o_ref,
               resid_ref, a_ref):
    i = pl.program_id(1)
    n = pl.program_id(2)
    rep = BT // 128

    @pl.when(jnp.logical_and(i == 0, n == 0))
    def _():
        x = jnp.maximum(hst_ref[...], 0.0)
        resid_ref[...] = x.reshape(NB, BN, BT)
        y = x * _rms_scale(x) * pltpu.repeat(nw0_ref[0], rep, axis=1)
        a_ref[...] = _qd_cols_bf16(y, ag_ref[0])

    zt = jax.lax.dot_general(
        wdq_ref[0], a_ref[...],
        dimension_numbers=(((1,), (0,)), ((), ())),
        preferred_element_type=jnp.float32,
    )
    resid_ref[n] = resid_ref[n] + zt * alpha_ref[i]

    @pl.when(n == NB - 1)
    def _():
        x = resid_ref[...].reshape(H, BT)
        y = x * _rms_scale(x) * pltpu.repeat(nwc_ref[0], rep, axis=1)

        @pl.when(i < 2)
        def _():
            a_ref[...] = _qd_cols_bf16(y, ag_ref[i + 1])

        @pl.when(i == 2)
        def _():
            o_ref[...] = y


def kernel(hidden_states, norm_w, w, agscale, wgscale):
    wdq = _prep_weights(w, wgscale)
    alpha = 1.0 / (wgscale * agscale)
    hst = hidden_states.T
    nwb = jnp.broadcast_to(norm_w[:, :, None], (4, H, 128))
    yt = pl.pallas_call(
        _main_body,
        grid=(T // BT, 3, NB),
        in_specs=[
            pl.BlockSpec(memory_space=pltpu.SMEM),
            pl.BlockSpec(memory_space=pltpu.SMEM),
            pl.BlockSpec((H, BT), lambda t, i, n: (0, t)),
            pl.BlockSpec((1, BN, H), lambda t, i, n: (i, n, 0)),
            pl.BlockSpec((1, H, 128), lambda t, i, n: (0, 0, 0)),
            pl.BlockSpec((1, H, 128), lambda t, i, n: (i + 1, 0, 0)),
        ],
        out_specs=pl.BlockSpec((H, BT), lambda t, i, n: (0, t)),
        out_shape=jax.ShapeDtypeStruct((H, T), jnp.float32),
        scratch_shapes=[
            pltpu.VMEM((NB, BN, BT), jnp.float32),
            pltpu.VMEM((H, BT), jnp.bfloat16),
        ],
        compiler_params=pltpu.CompilerParams(
            dimension_semantics=("parallel", "arbitrary", "arbitrary"),
            vmem_limit_bytes=100 * 1024 * 1024,
        ),
    )(agscale, alpha, hst, wdq, nwb, nwb)
    return yt.T


# speedup vs baseline: 1.3651x; 1.0293x over previous
import jax
import jax.numpy as jnp
from jax.experimental import pallas as pl
from jax.experimental.pallas import tpu as pltpu

H = 4096
T = 8192
EPS = 1e-06
BLK = 16
FP8_MAX = 448.0

BT = 256
BN = 1024
NB = H // BN
S = 4


def _fp8_e4m3(x):
    return x.astype(jnp.float8_e4m3fn).astype(jnp.float32)


def _fp4_round_mag(a):
    lo = jnp.floor(a + a + 0.5) * 0.5
    mid = jnp.floor(a + 0.5)
    hi = jnp.where(a >= 5.0, 6.0, 4.0)
    return jnp.where(a < 2.0, lo, jnp.where(a < 4.0, mid, hi))


def _qd_core(x, gs):
    amax = jnp.max(jnp.abs(x), axis=-2, keepdims=True)
    sc = jnp.clip(amax * (gs * (1.0 / 6.0)), 0.0, FP8_MAX)
    sc8 = _fp8_e4m3(sc)
    ok = sc8 > 0.0
    t = jnp.where(ok, (x * gs) / jnp.where(ok, sc8, 1.0), 0.0)
    qm = _fp4_round_mag(jnp.clip(jnp.abs(t), 0.0, 6.0))
    q = jnp.where(t < 0.0, -qm, qm)
    return q * sc8


def _qd_cols_bf16(xt, gs):
    xr = xt.reshape(H // BLK, BLK, BT)
    return _qd_core(xr, gs).reshape(H, BT).astype(jnp.bfloat16)


PBN = 512


def _roll_l(x, s):
    return jnp.concatenate([x[:, s:], x[:, :s]], axis=1)


def _roll_r(x, s):
    return jnp.concatenate([x[:, -s:], x[:, :-s]], axis=1)


def _seg16_max_lanes(a, lane_mod16):
    m = a
    for s in (1, 2, 4, 8):
        r = _roll_l(m, s)
        m = jnp.where(lane_mod16 < 16 - s, jnp.maximum(m, r), m)
    for s in (1, 2, 4, 8):
        r = _roll_r(m, s)
        m = jnp.where(lane_mod16 >= s, jnp.maximum(m, r), m)
    return m


def _prep_body(wg_ref, w_ref, o_ref):
    g = pl.program_id(0)
    i = g // (H // PBN)
    gs = wg_ref[i]
    x = w_ref[0]
    lane = jax.lax.broadcasted_iota(jnp.int32, x.shape, 1) % BLK
    amax = _seg16_max_lanes(jnp.abs(x), lane)
    sc = jnp.clip(amax * (gs * (1.0 / 6.0)), 0.0, FP8_MAX)
    sc8 = _fp8_e4m3(sc)
    ok = sc8 > 0.0
    t = jnp.where(ok, (x * gs) / jnp.where(ok, sc8, 1.0), 0.0)
    qm = _fp4_round_mag(jnp.clip(jnp.abs(t), 0.0, 6.0))
    q = jnp.where(t < 0.0, -qm, qm)
    o_ref[0] = (q * sc8).astype(jnp.bfloat16)


def _prep_weights(w, wgscale):
    return pl.pallas_call(
        _prep_body,
        grid=(3 * (H // PBN),),
        in_specs=[
            pl.BlockSpec(memory_space=pltpu.SMEM),
            pl.BlockSpec((1, PBN, H), lambda g: (g // (H // PBN), g % (H // PBN), 0)),
        ],
        out_specs=pl.BlockSpec((1, PBN, H), lambda g: (g // (H // PBN), g % (H // PBN), 0)),
        out_shape=jax.ShapeDtypeStruct((3, H, H), jnp.bfloat16),
        compiler_params=pltpu.CompilerParams(
            dimension_semantics=("parallel",),
            vmem_limit_bytes=100 * 1024 * 1024,
        ),
    )(wgscale, w)


def _rms_scale(xt):
    ms = jnp.sum(xt * xt, axis=0, keepdims=True) * (1.0 / H)
    return jax.lax.rsqrt(ms + EPS)


def _main_body(ag_ref, alpha_ref, hst_ref, wdq_ref, nw0_ref, nwc_ref, o_ref,
               resid_ref, a_ref, sem_in, sem_out):
    t = pl.program_id(0)
    i = pl.program_id(1)
    n = pl.program_id(2)
    s = pl.program_id(3)
    rep = BT // 128
    nto = pl.num_programs(0)

    def in_cp(sj):
        return pltpu.make_async_copy(
            hst_ref.at[:, :, pl.ds((t * S + sj) * BT, BT)],
            resid_ref.at[pl.ds(sj * NB, NB)],
            sem_in.at[sj])

    def out_cp(sj):
        return pltpu.make_async_copy(
            resid_ref.at[pl.ds(sj * NB, NB)],
            o_ref.at[:, :, pl.ds((t * S + sj) * BT, BT)],
            sem_out.at[sj])

    @pl.when(jnp.logical_and(jnp.logical_and(i == 0, n == 0), s == 0))
    def _():
        @pl.when(t > 0)
        def _():
            for sj in range(S):
                out_cp(sj).wait()
        for sj in range(S):
            in_cp(sj).start()

    soff = pl.multiple_of(s * NB, NB)

    @pl.when(jnp.logical_and(i == 0, n == 0))
    def _():
        in_cp(s).wait()
        x = jnp.maximum(resid_ref[pl.ds(soff, NB)], 0.0)
        resid_ref[pl.ds(soff, NB)] = x
        xf = x.reshape(H, BT)
        y = xf * _rms_scale(xf) * pltpu.repeat(nw0_ref[0], rep, axis=1)
        a_ref[s] = _qd_cols_bf16(y, ag_ref[0])

    zt = jax.lax.dot_general(
        wdq_ref[0], a_ref[s],
        dimension_numbers=(((1,), (0,)), ((), ())),
        preferred_element_type=jnp.float32,
    )
    resid_ref[soff + n] = resid_ref[soff + n] + zt * alpha_ref[i]

    @pl.when(n == NB - 1)
    def _():
        x = resid_ref[pl.ds(soff, NB)].reshape(H, BT)
        y = x * _rms_scale(x) * pltpu.repeat(nwc_ref[0], rep, axis=1)

        @pl.when(i < 2)
        def _():
            a_ref[s] = _qd_cols_bf16(y, ag_ref[i + 1])

        @pl.when(i == 2)
        def _():
            resid_ref[pl.ds(soff, NB)] = y.reshape(NB, BN, BT)
            out_cp(s).start()

            @pl.when(jnp.logical_and(t == nto - 1, s == S - 1))
            def _():
                for sj in range(S):
                    out_cp(sj).wait()


def kernel(hidden_states, norm_w, w, agscale, wgscale):
    wdq = _prep_weights(w, wgscale)
    alpha = 1.0 / (wgscale * agscale)
    hst3 = hidden_states.T.reshape(NB, BN, T)
    nwb = jnp.broadcast_to(norm_w[:, :, None], (4, H, 128))
    yt = pl.pallas_call(
        _main_body,
        grid=(T // (BT * S), 3, NB, S),
        in_specs=[
            pl.BlockSpec(memory_space=pltpu.SMEM),
            pl.BlockSpec(memory_space=pltpu.SMEM),
            pl.BlockSpec(memory_space=pl.ANY),
            pl.BlockSpec((1, BN, H), lambda t, i, n, s: (i, n, 0)),
            pl.BlockSpec((1, H, 128), lambda t, i, n, s: (0, 0, 0)),
            pl.BlockSpec((1, H, 128), lambda t, i, n, s: (i + 1, 0, 0)),
        ],
        out_specs=pl.BlockSpec(memory_space=pl.ANY),
        out_shape=jax.ShapeDtypeStruct((NB, BN, T), jnp.float32),
        scratch_shapes=[
            pltpu.VMEM((S * NB, BN, BT), jnp.float32),
            pltpu.VMEM((S, H, BT), jnp.bfloat16),
            pltpu.SemaphoreType.DMA((S,)),
            pltpu.SemaphoreType.DMA((S,)),
        ],
        compiler_params=pltpu.CompilerParams(
            dimension_semantics=("parallel", "arbitrary", "arbitrary", "arbitrary"),
            vmem_limit_bytes=100 * 1024 * 1024,
        ),
    )(agscale, alpha, hst3, wdq, nwb, nwb)
    return yt.reshape(H, T).T


# speedup vs baseline: 1.4889x; 1.0907x over previous
import jax
import jax.numpy as jnp
from jax.experimental import pallas as pl
from jax.experimental.pallas import tpu as pltpu

H = 4096
T = 8192
EPS = 1e-06
BLK = 16
FP8_MAX = 448.0

BT = 256
BN = 2048
NB = H // BN
S = 2


def _fp8_e4m3(x):
    return x.astype(jnp.float8_e4m3fn).astype(jnp.float32)


def _fp4_round_mag(a):
    lo = jnp.floor(a + a + 0.5) * 0.5
    mid = jnp.floor(a + 0.5)
    hi = jnp.where(a >= 5.0, 6.0, 4.0)
    return jnp.where(a < 2.0, lo, jnp.where(a < 4.0, mid, hi))


def _qd_core(x, gs):
    amax = jnp.max(jnp.abs(x), axis=-2, keepdims=True)
    sc = jnp.clip(amax * (gs * (1.0 / 6.0)), 0.0, FP8_MAX)
    sc8 = _fp8_e4m3(sc)
    ok = sc8 > 0.0
    rinv = jnp.where(ok, gs / jnp.where(ok, sc8, 1.0), 0.0)
    t = x * rinv
    qm = _fp4_round_mag(jnp.clip(jnp.abs(t), 0.0, 6.0))
    q = jnp.where(t < 0.0, -qm, qm)
    return q * sc8


def _qd_cols_bf16(xt, gs):
    xr = xt.reshape(H // BLK, BLK, BT)
    return _qd_core(xr, gs).reshape(H, BT).astype(jnp.bfloat16)


PBN = 512


def _prep_body(wg_ref, wt_ref, o_ref):
    g = pl.program_id(0)
    i = g // (H // PBN)
    x = wt_ref[0].reshape(H // BLK, BLK, PBN)
    o_ref[0] = _qd_core(x, wg_ref[i]).reshape(H, PBN).astype(jnp.bfloat16)


def _prep_weights(w, wgscale):
    wt = w.transpose(0, 2, 1)
    return pl.pallas_call(
        _prep_body,
        grid=(3 * (H // PBN),),
        in_specs=[
            pl.BlockSpec(memory_space=pltpu.SMEM),
            pl.BlockSpec((1, H, PBN), lambda g: (g // (H // PBN), 0, g % (H // PBN))),
        ],
        out_specs=pl.BlockSpec((1, H, PBN), lambda g: (g // (H // PBN), 0, g % (H // PBN))),
        out_shape=jax.ShapeDtypeStruct((3, H, H), jnp.bfloat16),
        compiler_params=pltpu.CompilerParams(
            dimension_semantics=("parallel",),
            vmem_limit_bytes=100 * 1024 * 1024,
        ),
    )(wgscale, wt)


def _rms_scale(xt):
    ms = jnp.sum(xt * xt, axis=0, keepdims=True) * (1.0 / H)
    return jax.lax.rsqrt(ms + EPS)


def _main_body(ag_ref, alpha_ref, hst_ref, wdq_ref, nw_ref, o_ref,
               resid_ref, a_ref, sem_in, sem_out):
    t = pl.program_id(0)
    i = pl.program_id(1)
    n = pl.program_id(2)
    s = pl.program_id(3)
    rep = BT // 128
    nto = pl.num_programs(0)

    def in_cp(sj):
        return pltpu.make_async_copy(
            hst_ref.at[:, :, pl.ds((t * S + sj) * BT, BT)],
            resid_ref.at[pl.ds(sj * NB, NB)],
            sem_in.at[sj])

    def out_cp(sj):
        return pltpu.make_async_copy(
            resid_ref.at[pl.ds(sj * NB, NB)],
            o_ref.at[:, :, pl.ds((t * S + sj) * BT, BT)],
            sem_out.at[sj])

    @pl.when(jnp.logical_and(jnp.logical_and(i == 0, n == 0), s == 0))
    def _():
        @pl.when(t > 0)
        def _():
            for sj in range(S):
                out_cp(sj).wait()
        for sj in range(S):
            in_cp(sj).start()

    soff = pl.multiple_of(s * NB, NB)

    @pl.when(jnp.logical_and(i == 0, n == 0))
    def _():
        in_cp(s).wait()
        x = jnp.maximum(resid_ref[pl.ds(soff, NB)], 0.0)
        resid_ref[pl.ds(soff, NB)] = x
        xf = x.reshape(H, BT)
        y = xf * _rms_scale(xf) * pltpu.repeat(nw_ref[0], rep, axis=1)
        a_ref[s] = _qd_cols_bf16(y, ag_ref[0])

    zt = jax.lax.dot_general(
        wdq_ref[0], a_ref[s],
        dimension_numbers=(((0,), (0,)), ((), ())),
        preferred_element_type=jnp.float32,
    )
    resid_ref[soff + n] = resid_ref[soff + n] + zt * alpha_ref[i]

    @pl.when(n == NB - 1)
    def _():
        x = resid_ref[pl.ds(soff, NB)].reshape(H, BT)
        y = x * _rms_scale(x) * pltpu.repeat(nw_ref[0], rep, axis=1)

        @pl.when(i < 2)
        def _():
            a_ref[s] = _qd_cols_bf16(y, ag_ref[i + 1])

        @pl.when(i == 2)
        def _():
            resid_ref[pl.ds(soff, NB)] = y.reshape(NB, BN, BT)
            out_cp(s).start()

            @pl.when(jnp.logical_and(t == nto - 1, s == S - 1))
            def _():
                for sj in range(S):
                    out_cp(sj).wait()


def kernel(hidden_states, norm_w, w, agscale, wgscale):
    wdq = _prep_weights(w, wgscale)
    alpha = 1.0 / (wgscale * agscale)
    hst3 = hidden_states.T.reshape(NB, BN, T)
    nwb = jnp.broadcast_to(norm_w[:, :, None], (4, H, 128))
    yt = pl.pallas_call(
        _main_body,
        grid=(T // (BT * S), 3, NB, S),
        in_specs=[
            pl.BlockSpec(memory_space=pltpu.SMEM),
            pl.BlockSpec(memory_space=pltpu.SMEM),
            pl.BlockSpec(memory_space=pl.ANY),
            pl.BlockSpec((1, H, BN), lambda t, i, n, s: (i, 0, n)),
            pl.BlockSpec((1, H, 128),
                         lambda t, i, n, s: (jnp.where(n == 0, 0, i + 1), 0, 0)),
        ],
        out_specs=pl.BlockSpec(memory_space=pl.ANY),
        out_shape=jax.ShapeDtypeStruct((NB, BN, T), jnp.float32),
        scratch_shapes=[
            pltpu.VMEM((S * NB, BN, BT), jnp.float32),
            pltpu.VMEM((S, H, BT), jnp.bfloat16),
            pltpu.SemaphoreType.DMA((S,)),
            pltpu.SemaphoreType.DMA((S,)),
        ],
        compiler_params=pltpu.CompilerParams(
            dimension_semantics=("parallel", "arbitrary", "arbitrary", "arbitrary"),
            vmem_limit_bytes=100 * 1024 * 1024,
        ),
    )(agscale, alpha, hst3, wdq, nwb)
    return yt.reshape(H, T).T


# speedup vs baseline: 1.6094x; 1.0809x over previous
import jax
import jax.numpy as jnp
from jax.experimental import pallas as pl
from jax.experimental.pallas import tpu as pltpu

H = 4096
T = 8192
EPS = 1e-06
BLK = 16
FP8_MAX = 448.0

BT = 256
BN = 2048
NB = H // BN
S = 2


def _fp8_e4m3(x):
    return x.astype(jnp.float8_e4m3fn).astype(jnp.float32)


def _fp4_round_mag(a):
    lo = jnp.floor(a + a + 0.5) * 0.5
    mid = jnp.floor(a + 0.5)
    hi = jnp.where(a >= 5.0, 6.0, 4.0)
    return jnp.where(a < 2.0, lo, jnp.where(a < 4.0, mid, hi))


def _qd_core(x, gs):
    amax = jnp.max(jnp.abs(x), axis=-2, keepdims=True)
    sc = jnp.clip(amax * (gs * (1.0 / 6.0)), 0.0, FP8_MAX)
    sc8 = _fp8_e4m3(sc)
    ok = sc8 > 0.0
    rinv = jnp.where(ok, gs / jnp.where(ok, sc8, 1.0), 0.0)
    t = x * rinv
    qm = _fp4_round_mag(jnp.clip(jnp.abs(t), 0.0, 6.0))
    q = jnp.where(t < 0.0, -qm, qm)
    return q * sc8


def _qd_cols_bf16(xt, gs):
    xr = xt.reshape(H // BLK, BLK, BT)
    return _qd_core(xr, gs).reshape(H, BT).astype(jnp.bfloat16)


PBN = 512


def _prep_body(wg_ref, w_ref, o_ref):
    g = pl.program_id(0)
    i = g // (H // PBN)
    xt = jnp.swapaxes(w_ref[0], 0, 1)
    x = xt.reshape(PBN // BLK, BLK, H)
    o_ref[0] = _qd_core(x, wg_ref[i]).reshape(PBN, H).astype(jnp.bfloat16)


def _prep_weights(w, wgscale):
    return pl.pallas_call(
        _prep_body,
        grid=(3 * (H // PBN),),
        in_specs=[
            pl.BlockSpec(memory_space=pltpu.SMEM),
            pl.BlockSpec((1, H, PBN), lambda g: (g // (H // PBN), 0, g % (H // PBN))),
        ],
        out_specs=pl.BlockSpec((1, PBN, H), lambda g: (g // (H // PBN), g % (H // PBN), 0)),
        out_shape=jax.ShapeDtypeStruct((3, H, H), jnp.bfloat16),
        compiler_params=pltpu.CompilerParams(
            dimension_semantics=("parallel",),
            vmem_limit_bytes=100 * 1024 * 1024,
        ),
    )(wgscale, w)


def _rms_scale(xt):
    ms = jnp.sum(xt * xt, axis=0, keepdims=True) * (1.0 / H)
    return jax.lax.rsqrt(ms + EPS)


def _main_body(ag_ref, alpha_ref, hst_ref, wdq_ref, nw_ref, o_ref,
               resid_ref, a_ref, sem_in, sem_out):
    t = pl.program_id(0)
    i = pl.program_id(1)
    n = pl.program_id(2)
    s = pl.program_id(3)
    rep = BT // 128
    nto = pl.num_programs(0)

    def in_cp(sj):
        return pltpu.make_async_copy(
            hst_ref.at[:, :, pl.ds((t * S + sj) * BT, BT)],
            resid_ref.at[pl.ds(sj * NB, NB)],
            sem_in.at[sj])

    def out_cp(sj):
        return pltpu.make_async_copy(
            resid_ref.at[pl.ds(sj * NB, NB)],
            o_ref.at[:, :, pl.ds((t * S + sj) * BT, BT)],
            sem_out.at[sj])

    @pl.when(jnp.logical_and(jnp.logical_and(i == 0, n == 0), s == 0))
    def _():
        @pl.when(t > 0)
        def _():
            for sj in range(S):
                out_cp(sj).wait()
        for sj in range(S):
            in_cp(sj).start()

    soff = pl.multiple_of(s * NB, NB)

    @pl.when(jnp.logical_and(i == 0, n == 0))
    def _():
        in_cp(s).wait()
        x = jnp.maximum(resid_ref[pl.ds(soff, NB)], 0.0)
        resid_ref[pl.ds(soff, NB)] = x
        xf = x.reshape(H, BT)
        y = xf * _rms_scale(xf) * pltpu.repeat(nw_ref[0], rep, axis=1)
        a_ref[s] = _qd_cols_bf16(y, ag_ref[0])

    zt = jax.lax.dot_general(
        wdq_ref[0], a_ref[s],
        dimension_numbers=(((0,), (0,)), ((), ())),
        preferred_element_type=jnp.float32,
    )
    resid_ref[soff + n] = resid_ref[soff + n] + zt * alpha_ref[i]

    @pl.when(n == NB - 1)
    def _():
        x = resid_ref[pl.ds(soff, NB)].reshape(H, BT)
        y = x * _rms_scale(x) * pltpu.repeat(nw_ref[0], rep, axis=1)

        @pl.when(i < 2)
        def _():
            a_ref[s] = _qd_cols_bf16(y, ag_ref[i + 1])

        @pl.when(i == 2)
        def _():
            resid_ref[pl.ds(soff, NB)] = y.reshape(NB, BN, BT)
            out_cp(s).start()

            @pl.when(jnp.logical_and(t == nto - 1, s == S - 1))
            def _():
                for sj in range(S):
                    out_cp(sj).wait()


def kernel(hidden_states, norm_w, w, agscale, wgscale):
    wdq = _prep_weights(w, wgscale)
    alpha = 1.0 / (wgscale * agscale)
    hst3 = hidden_states.T.reshape(NB, BN, T)
    nwb = jnp.broadcast_to(norm_w[:, :, None], (4, H, 128))
    yt = pl.pallas_call(
        _main_body,
        grid=(T // (BT * S), 3, NB, S),
        in_specs=[
            pl.BlockSpec(memory_space=pltpu.SMEM),
            pl.BlockSpec(memory_space=pltpu.SMEM),
            pl.BlockSpec(memory_space=pl.ANY),
            pl.BlockSpec((1, H, BN), lambda t, i, n, s: (i, 0, n)),
            pl.BlockSpec((1, H, 128),
                         lambda t, i, n, s: (jnp.where(jnp.logical_and(i == 0, n == 0),
                                                       0, i + 1), 0, 0)),
        ],
        out_specs=pl.BlockSpec(memory_space=pl.ANY),
        out_shape=jax.ShapeDtypeStruct((NB, BN, T), jnp.float32),
        scratch_shapes=[
            pltpu.VMEM((S * NB, BN, BT), jnp.float32),
            pltpu.VMEM((S, H, BT), jnp.bfloat16),
            pltpu.SemaphoreType.DMA((S,)),
            pltpu.SemaphoreType.DMA((S,)),
        ],
        compiler_params=pltpu.CompilerParams(
            dimension_semantics=("parallel", "arbitrary", "arbitrary", "arbitrary"),
            vmem_limit_bytes=100 * 1024 * 1024,
        ),
    )(agscale, alpha, hst3, wdq, nwb)
    return yt.reshape(H, T).T
